# baseline (speedup 1.0000x reference)
import functools
import math

import jax
import jax.numpy as jnp
from jax import lax
from jax.experimental import pallas as pl
from jax.experimental.pallas import tpu as pltpu
from jax.experimental.pallas import tpu_sc as plsc

_N = 1024
_V = 100000
_PAD = 0
_SMOOTH_VAL = 0.1 / (_V - 2)
_CONF = 0.9
_K = (_V - 2) * _SMOOTH_VAL * math.log(_SMOOTH_VAL) + _CONF * math.log(_CONF)

_NC, _NS, _L = 2, 16, 16
_NW = _NC * _NS
_RPW = _N // _NW
_NVEC = _RPW // _L


def _sc_body(x_hbm, t_hbm, out_hbm, t_v, idx_v, g_v, acc_v, sem):
    wid = lax.axis_index("s") * _NC + lax.axis_index("c")
    base = wid * _RPW
    pltpu.sync_copy(t_hbm.at[pl.ds(base, _RPW)], t_v)
    for j in range(_NVEC):
        t16 = t_v[pl.ds(j * _L, _L)]
        rows = (base + j * _L) + lax.iota(jnp.int32, _L)
        idx_v[pl.ds(j * _L, _L)] = rows * _V + t16
        idx_v[pl.ds(_RPW + j * _L, _L)] = rows * _V
    pltpu.async_copy(x_hbm.at[idx_v], g_v, sem).wait()
    acc = jnp.zeros((_L,), jnp.float32)
    for j in range(_NVEC):
        t16 = t_v[pl.ds(j * _L, _L)]
        gt = g_v[pl.ds(j * _L, _L)]
        g0 = g_v[pl.ds(_RPW + j * _L, _L)]
        term = _K + _SMOOTH_VAL * g0 + (_SMOOTH_VAL - _CONF) * gt
        acc = acc + jnp.where(t16 != _PAD, term, 0.0)
    acc_v[...] = acc
    pltpu.sync_copy(acc_v, out_hbm.at[wid])


@functools.cache
def _sc_call():
    return functools.partial(
        pl.kernel,
        out_type=jax.ShapeDtypeStruct((_NW, _L), jnp.float32),
        mesh=plsc.VectorSubcoreMesh(core_axis_name="c", subcore_axis_name="s"),
        scratch_types=[
            pltpu.VMEM((_RPW,), jnp.int32),
            pltpu.VMEM((2 * _RPW,), jnp.int32),
            pltpu.VMEM((2 * _RPW,), jnp.float32),
            pltpu.VMEM((_L,), jnp.float32),
            pltpu.SemaphoreType.DMA,
        ],
    )(_sc_body)


_BC = 2048
_GRID = -(-_V // _BC)


def _tc_body(w_ref, x_ref, out_ref):
    j = pl.program_id(0)

    @pl.when(j == 0)
    def _():
        out_ref[...] = jnp.zeros_like(out_ref)

    rv = lax.dot_general(w_ref[...], x_ref[...], (((1,), (0,)), ((), ())),
                         preferred_element_type=jnp.float32)
    col = j * _BC + lax.broadcasted_iota(jnp.int32, (1, _BC), 1)
    rv = jnp.where(col < _V, rv, 0.0)
    out_ref[...] += jnp.sum(rv).reshape(1, 1)


_tc_call = pl.pallas_call(
    _tc_body,
    grid=(_GRID,),
    in_specs=[
        pl.BlockSpec((1, _N), lambda j: (0, 0)),
        pl.BlockSpec((_N, _BC), lambda j: (0, j)),
    ],
    out_specs=pl.BlockSpec((1, 1), lambda j: (0, 0)),
    out_shape=jax.ShapeDtypeStruct((1, 1), jnp.float32),
)


def kernel(x, target):
    assert x.shape == (_N, _V)
    t32 = target.astype(jnp.int32)
    sc_out = _sc_call()(x.reshape(-1), t32)
    w = (t32 != _PAD).astype(jnp.float32).reshape(1, _N)
    tc_out = _tc_call(w, x)
    return jnp.sum(sc_out) - jnp.float32(_SMOOTH_VAL) * tc_out[0, 0]


# speedup vs baseline: 2.0971x; 2.0971x over previous
import functools
import math

import jax
import jax.numpy as jnp
from jax import lax
from jax.experimental import pallas as pl
from jax.experimental.pallas import tpu as pltpu
from jax.experimental.pallas import tpu_sc as plsc

_N = 1024
_V = 100000
_PAD = 0
_SMOOTH_VAL = 0.1 / (_V - 2)
_CONF = 0.9
_K = (_V - 2) * _SMOOTH_VAL * math.log(_SMOOTH_VAL) + _CONF * math.log(_CONF)

_NC, _NS, _L = 2, 16, 16
_NW = _NC * _NS
_RPW = _N // _NW
_NVEC = _RPW // _L


def _sc_body(t_hbm, w_hbm, t_v, w_v):
    wid = lax.axis_index("s") * _NC + lax.axis_index("c")
    base = wid * _RPW
    pltpu.sync_copy(t_hbm.at[pl.ds(base, _RPW)], t_v)
    for j in range(_NVEC):
        t16 = t_v[pl.ds(j * _L, _L)]
        w_v[pl.ds(j * _L, _L)] = jnp.where(t16 != _PAD, 1.0, 0.0)
    pltpu.sync_copy(w_v, w_hbm.at[pl.ds(base, _RPW)])


@functools.cache
def _sc_call():
    return functools.partial(
        pl.kernel,
        out_type=jax.ShapeDtypeStruct((_N,), jnp.float32),
        mesh=plsc.VectorSubcoreMesh(core_axis_name="c", subcore_axis_name="s"),
        scratch_types=[
            pltpu.VMEM((_RPW,), jnp.int32),
            pltpu.VMEM((_RPW,), jnp.float32),
        ],
    )(_sc_body)


_BC = 2048
_GRID = -(-_V // _BC)


def _tc_body(w_ref, t_ref, x_ref, out_ref):
    j = pl.program_id(0)
    xb = x_ref[...]
    w = w_ref[...]

    @pl.when(j == 0)
    def _():
        nv = jnp.sum(w)
        col0 = lax.dot_general(w, xb[:, 0:1], (((1,), (0,)), ((), ())),
                               preferred_element_type=jnp.float32)
        out_ref[...] = (_K * nv + _SMOOTH_VAL * col0[0, 0]).reshape(1, 1)

    col = j * _BC + lax.broadcasted_iota(jnp.int32, (_N, _BC), 1)
    pick = jnp.where(col == t_ref[...], xb, 0.0)
    bulk = lax.dot_general(w, xb, (((1,), (0,)), ((), ())),
                           preferred_element_type=jnp.float32)
    eqv = lax.dot_general(w, pick, (((1,), (0,)), ((), ())),
                          preferred_element_type=jnp.float32)
    vec = (_SMOOTH_VAL - _CONF) * eqv - _SMOOTH_VAL * bulk
    cvec = j * _BC + lax.broadcasted_iota(jnp.int32, (1, _BC), 1)
    out_ref[...] += jnp.sum(jnp.where(cvec < _V, vec, 0.0)).reshape(1, 1)


_tc_call = pl.pallas_call(
    _tc_body,
    grid=(_GRID,),
    in_specs=[
        pl.BlockSpec((1, _N), lambda j: (0, 0)),
        pl.BlockSpec((_N, 1), lambda j: (0, 0)),
        pl.BlockSpec((_N, _BC), lambda j: (0, j)),
    ],
    out_specs=pl.BlockSpec((1, 1), lambda j: (0, 0)),
    out_shape=jax.ShapeDtypeStruct((1, 1), jnp.float32),
)


def kernel(x, target):
    assert x.shape == (_N, _V)
    t32 = target.astype(jnp.int32)
    w = _sc_call()(t32).reshape(1, _N)
    tc_out = _tc_call(w, t32.reshape(_N, 1), x)
    return tc_out[0, 0]


# speedup vs baseline: 2.1566x; 1.0284x over previous
import functools
import math

import jax
import jax.numpy as jnp
from jax import lax
from jax.experimental import pallas as pl
from jax.experimental.pallas import tpu as pltpu
from jax.experimental.pallas import tpu_sc as plsc

_N = 1024
_V = 100000
_PAD = 0
_SMOOTH_VAL = 0.1 / (_V - 2)
_CONF = 0.9
_K = (_V - 2) * _SMOOTH_VAL * math.log(_SMOOTH_VAL) + _CONF * math.log(_CONF)

_NC, _NS, _L = 2, 16, 16
_NW = _NC * _NS
_RPW = _N // _NW
_NVEC = _RPW // _L


def _sc_body(t_hbm, w_hbm, t_v, w_v):
    wid = lax.axis_index("s") * _NC + lax.axis_index("c")
    base = wid * _RPW
    pltpu.sync_copy(t_hbm.at[pl.ds(base, _RPW)], t_v)
    for j in range(_NVEC):
        t16 = t_v[pl.ds(j * _L, _L)]
        w_v[pl.ds(j * _L, _L)] = jnp.where(t16 != _PAD, 1.0, 0.0)
    pltpu.sync_copy(w_v, w_hbm.at[pl.ds(base, _RPW)])


@functools.cache
def _sc_call():
    return functools.partial(
        pl.kernel,
        out_type=jax.ShapeDtypeStruct((_N,), jnp.float32),
        mesh=plsc.VectorSubcoreMesh(core_axis_name="c", subcore_axis_name="s"),
        scratch_types=[
            pltpu.VMEM((_RPW,), jnp.int32),
            pltpu.VMEM((_RPW,), jnp.float32),
        ],
    )(_sc_body)


_BC = 2048
_GRID = -(-_V // _BC)


def _tc_body(w_ref, t_ref, x_ref, out_ref):
    j = pl.program_id(0)
    xb = x_ref[...]
    w = w_ref[...]

    @pl.when(j == 0)
    def _():
        nv = jnp.sum(w)
        col0 = lax.dot_general(w, xb[:, 0:1], (((1,), (0,)), ((), ())),
                               preferred_element_type=jnp.float32)
        out_ref[...] = (_K * nv + _SMOOTH_VAL * col0[0, 0]).reshape(1, 1)

    col = j * _BC + lax.broadcasted_iota(jnp.int32, (_N, _BC), 1)
    pick = jnp.where(col == t_ref[...], xb, 0.0)
    bulk = lax.dot_general(w, xb, (((1,), (0,)), ((), ())),
                           preferred_element_type=jnp.float32)
    eqv = lax.dot_general(w, pick, (((1,), (0,)), ((), ())),
                          preferred_element_type=jnp.float32)
    vec = (_SMOOTH_VAL - _CONF) * eqv - _SMOOTH_VAL * bulk
    cvec = j * _BC + lax.broadcasted_iota(jnp.int32, (1, _BC), 1)
    out_ref[...] += jnp.sum(jnp.where(cvec < _V, vec, 0.0)).reshape(1, 1)


_tc_call = pl.pallas_call(
    _tc_body,
    grid=(_GRID,),
    in_specs=[
        pl.BlockSpec((1, _N), lambda j: (0, 0)),
        pl.BlockSpec((_N, 1), lambda j: (0, 0)),
        pl.BlockSpec((_N, _BC), lambda j: (0, j)),
    ],
    out_specs=pl.BlockSpec((1, 1), lambda j: (0, 0)),
    out_shape=jax.ShapeDtypeStruct((1, 1), jnp.float32),
)


def kernel(x, target):
    assert x.shape == (_N, _V)
    t32 = target.astype(jnp.int32)
    w = (t32 != _PAD).astype(jnp.float32).reshape(1, _N)
    tc_out = _tc_call(w, t32.reshape(_N, 1), x)
    return tc_out[0, 0]


# speedup vs baseline: 6.0796x; 2.8191x over previous
import functools
import math

import jax
import jax.numpy as jnp
from jax import lax
from jax.experimental import pallas as pl
from jax.experimental.pallas import tpu as pltpu
from jax.experimental.pallas import tpu_sc as plsc

_N = 1024
_V = 100000
_PAD = 0
_SMOOTH_VAL = 0.1 / (_V - 2)
_CONF = 0.9
_K = (_V - 2) * _SMOOTH_VAL * math.log(_SMOOTH_VAL) + _CONF * math.log(_CONF)

_NC, _NS, _L = 2, 16, 16
_NW = _NC * _NS
_RPW = _N // _NW
_NVEC = _RPW // _L


def _sc_body(t_hbm, w_hbm, t_v, w_v):
    wid = lax.axis_index("s") * _NC + lax.axis_index("c")
    base = wid * _RPW
    pltpu.sync_copy(t_hbm.at[pl.ds(base, _RPW)], t_v)
    for j in range(_NVEC):
        t16 = t_v[pl.ds(j * _L, _L)]
        w_v[pl.ds(j * _L, _L)] = jnp.where(t16 != _PAD, 1.0, 0.0)
    pltpu.sync_copy(w_v, w_hbm.at[pl.ds(base, _RPW)])


@functools.cache
def _sc_call():
    return functools.partial(
        pl.kernel,
        out_type=jax.ShapeDtypeStruct((_N,), jnp.float32),
        mesh=plsc.VectorSubcoreMesh(core_axis_name="c", subcore_axis_name="s"),
        scratch_types=[
            pltpu.VMEM((_RPW,), jnp.int32),
            pltpu.VMEM((_RPW,), jnp.float32),
        ],
    )(_sc_body)


_BR = 2048
_GRID = -(-_V // _BR)


def _tc_body(w_ref, t_ref, x_ref, out_ref):
    j = pl.program_id(0)
    xb = x_ref[...]
    w = w_ref[...]

    @pl.when(j == 0)
    def _():
        nv = jnp.sum(w)
        row0 = lax.dot_general(xb[0:1, :], w, (((1,), (1,)), ((), ())),
                               preferred_element_type=jnp.float32)
        out_ref[...] = (_K * nv + _SMOOTH_VAL * row0[0, 0]).reshape(1, 1)

    vcol = j * _BR + lax.broadcasted_iota(jnp.int32, (_BR, _N), 0)
    pick = jnp.where(vcol == t_ref[...], xb, 0.0)
    bulk = lax.dot_general(xb, w, (((1,), (1,)), ((), ())),
                           preferred_element_type=jnp.float32)
    eqv = lax.dot_general(pick, w, (((1,), (1,)), ((), ())),
                          preferred_element_type=jnp.float32)
    vec = (_SMOOTH_VAL - _CONF) * eqv - _SMOOTH_VAL * bulk
    vrow = j * _BR + lax.broadcasted_iota(jnp.int32, (_BR, 1), 0)
    out_ref[...] += jnp.sum(jnp.where(vrow < _V, vec, 0.0)).reshape(1, 1)


_tc_call = pl.pallas_call(
    _tc_body,
    grid=(_GRID,),
    in_specs=[
        pl.BlockSpec((1, _N), lambda j: (0, 0)),
        pl.BlockSpec((1, _N), lambda j: (0, 0)),
        pl.BlockSpec((_BR, _N), lambda j: (j, 0)),
    ],
    out_specs=pl.BlockSpec((1, 1), lambda j: (0, 0)),
    out_shape=jax.ShapeDtypeStruct((1, 1), jnp.float32),
)


def kernel(x, target):
    assert x.shape == (_N, _V)
    t32 = target.astype(jnp.int32)
    w = _sc_call()(t32).reshape(1, _N)
    tc_out = _tc_call(w, t32.reshape(1, _N), jnp.swapaxes(x, 0, 1))
    return tc_out[0, 0]


# speedup vs baseline: 6.1847x; 1.0173x over previous
import functools
import math

import jax
import jax.numpy as jnp
from jax import lax
from jax.experimental import pallas as pl
from jax.experimental.pallas import tpu as pltpu
from jax.experimental.pallas import tpu_sc as plsc

_N = 1024
_V = 100000
_PAD = 0
_SMOOTH_VAL = 0.1 / (_V - 2)
_CONF = 0.9
_K = (_V - 2) * _SMOOTH_VAL * math.log(_SMOOTH_VAL) + _CONF * math.log(_CONF)

_NC, _NS, _L = 2, 16, 16
_NW = _NC * _NS
_RPW = _N // _NW
_NVEC = _RPW // _L


def _sc_body(t_hbm, out_hbm, t_v, acc_v):
    wid = lax.axis_index("s") * _NC + lax.axis_index("c")
    base = wid * _RPW
    pltpu.sync_copy(t_hbm.at[pl.ds(base, _RPW)], t_v)
    acc = jnp.zeros((_L,), jnp.float32)
    for j in range(_NVEC):
        t16 = t_v[pl.ds(j * _L, _L)]
        acc = acc + jnp.where(t16 != _PAD, _K, 0.0)
    acc_v[...] = acc
    pltpu.sync_copy(acc_v, out_hbm.at[wid])


@functools.cache
def _sc_call():
    return functools.partial(
        pl.kernel,
        out_type=jax.ShapeDtypeStruct((_NW, _L), jnp.float32),
        mesh=plsc.VectorSubcoreMesh(core_axis_name="c", subcore_axis_name="s"),
        scratch_types=[
            pltpu.VMEM((_RPW,), jnp.int32),
            pltpu.VMEM((_L,), jnp.float32),
        ],
    )(_sc_body)


_BR = 2048
_GRID = -(-_V // _BR)


def _tc_body(t_ref, x_ref, out_ref):
    j = pl.program_id(0)
    xb = x_ref[...]
    t = t_ref[...]
    w = jnp.where(t != _PAD, 1.0, 0.0)

    @pl.when(j == 0)
    def _():
        row0 = lax.dot_general(xb[0:1, :], w, (((1,), (1,)), ((), ())),
                               preferred_element_type=jnp.float32)
        out_ref[...] = (_SMOOTH_VAL * row0[0, 0]).reshape(1, 1)

    vcol = j * _BR + lax.broadcasted_iota(jnp.int32, (_BR, _N), 0)
    pick = jnp.where(vcol == t, xb, 0.0)
    bulk = lax.dot_general(xb, w, (((1,), (1,)), ((), ())),
                           preferred_element_type=jnp.float32)
    eqv = lax.dot_general(pick, w, (((1,), (1,)), ((), ())),
                          preferred_element_type=jnp.float32)
    vec = (_SMOOTH_VAL - _CONF) * eqv - _SMOOTH_VAL * bulk
    vrow = j * _BR + lax.broadcasted_iota(jnp.int32, (_BR, 1), 0)
    out_ref[...] += jnp.sum(jnp.where(vrow < _V, vec, 0.0)).reshape(1, 1)


_tc_call = pl.pallas_call(
    _tc_body,
    grid=(_GRID,),
    in_specs=[
        pl.BlockSpec((1, _N), lambda j: (0, 0)),
        pl.BlockSpec((_BR, _N), lambda j: (j, 0)),
    ],
    out_specs=pl.BlockSpec((1, 1), lambda j: (0, 0)),
    out_shape=jax.ShapeDtypeStruct((1, 1), jnp.float32),
)


def kernel(x, target):
    assert x.shape == (_N, _V)
    t32 = target.astype(jnp.int32)
    k_partials = _sc_call()(t32)
    tc_out = _tc_call(t32.reshape(1, _N), jnp.swapaxes(x, 0, 1))
    return tc_out[0, 0] + jnp.sum(k_partials)


# speedup vs baseline: 6.5175x; 1.0538x over previous
import functools
import math

import jax
import jax.numpy as jnp
from jax import lax
from jax.experimental import pallas as pl
from jax.experimental.pallas import tpu as pltpu
from jax.experimental.pallas import tpu_sc as plsc

_N = 1024
_V = 100000
_PAD = 0
_SMOOTH_VAL = 0.1 / (_V - 2)
_CONF = 0.9
_K = (_V - 2) * _SMOOTH_VAL * math.log(_SMOOTH_VAL) + _CONF * math.log(_CONF)

_NC, _NS, _L = 2, 16, 16
_NW = _NC * _NS
_RPW = _N // _NW
_NVEC = _RPW // _L


def _sc_body(t_hbm, out_hbm, t_v, acc_v):
    wid = lax.axis_index("s") * _NC + lax.axis_index("c")
    base = wid * _RPW
    pltpu.sync_copy(t_hbm.at[pl.ds(base, _RPW)], t_v)
    acc = jnp.zeros((_L,), jnp.float32)
    for j in range(_NVEC):
        t16 = t_v[pl.ds(j * _L, _L)]
        acc = acc + jnp.where(t16 != _PAD, _K, 0.0)
    acc_v[...] = acc
    pltpu.sync_copy(acc_v, out_hbm.at[wid])


@functools.cache
def _sc_call():
    return functools.partial(
        pl.kernel,
        out_type=jax.ShapeDtypeStruct((_NW, _L), jnp.float32),
        mesh=plsc.VectorSubcoreMesh(core_axis_name="c", subcore_axis_name="s"),
        scratch_types=[
            pltpu.VMEM((_RPW,), jnp.int32),
            pltpu.VMEM((_L,), jnp.float32),
        ],
    )(_sc_body)


_BR = 2048
_GRID = -(-_V // _BR)


def _tc_body(t_ref, tc_ref, x_ref, out_ref):
    j = pl.program_id(0)
    xb = x_ref[...]
    w = jnp.where(tc_ref[...] != _PAD, 1.0, 0.0)

    @pl.when(j == 0)
    def _():
        row0 = lax.dot_general(xb[0:1, :], w, (((1,), (0,)), ((), ())),
                               preferred_element_type=jnp.float32)
        out_ref[...] = (_SMOOTH_VAL * row0[0, 0]).reshape(1, 1)

    vcol = j * _BR + lax.broadcasted_iota(jnp.int32, (_BR, _N), 0)
    y = xb * jnp.where(vcol == t_ref[...], -_CONF, -_SMOOTH_VAL)
    vec = lax.dot_general(y, w, (((1,), (0,)), ((), ())),
                          preferred_element_type=jnp.float32)
    vrow = j * _BR + lax.broadcasted_iota(jnp.int32, (_BR, 1), 0)
    out_ref[...] += jnp.sum(jnp.where(vrow < _V, vec, 0.0)).reshape(1, 1)


_tc_call = pl.pallas_call(
    _tc_body,
    grid=(_GRID,),
    in_specs=[
        pl.BlockSpec((1, _N), lambda j: (0, 0)),
        pl.BlockSpec((_N, 1), lambda j: (0, 0)),
        pl.BlockSpec((_BR, _N), lambda j: (j, 0)),
    ],
    out_specs=pl.BlockSpec((1, 1), lambda j: (0, 0)),
    out_shape=jax.ShapeDtypeStruct((1, 1), jnp.float32),
)


def kernel(x, target):
    assert x.shape == (_N, _V)
    t32 = target.astype(jnp.int32)
    k_partials = _sc_call()(t32)
    tc_out = _tc_call(t32.reshape(1, _N), t32.reshape(_N, 1),
                      jnp.swapaxes(x, 0, 1))
    return tc_out[0, 0] + jnp.sum(k_partials)


# speedup vs baseline: 7.0494x; 1.0816x over previous
import functools
import math

import jax
import jax.numpy as jnp
from jax import lax
from jax.experimental import pallas as pl
from jax.experimental.pallas import tpu as pltpu
from jax.experimental.pallas import tpu_sc as plsc

_N = 1024
_V = 100000
_PAD = 0
_SMOOTH_VAL = 0.1 / (_V - 2)
_CONF = 0.9
_K = (_V - 2) * _SMOOTH_VAL * math.log(_SMOOTH_VAL) + _CONF * math.log(_CONF)

_NC, _NS, _L = 2, 16, 16
_NW = _NC * _NS
_RPW = _N // _NW
_NVEC = _RPW // _L


def _sc_body(t_hbm, out_hbm, t_v, acc_v):
    wid = lax.axis_index("s") * _NC + lax.axis_index("c")
    base = wid * _RPW
    pltpu.sync_copy(t_hbm.at[pl.ds(base, _RPW)], t_v)
    acc = jnp.zeros((_L,), jnp.float32)
    for j in range(_NVEC):
        t16 = t_v[pl.ds(j * _L, _L)]
        acc = acc + jnp.where(t16 != _PAD, _K, 0.0)
    acc_v[...] = acc
    pltpu.sync_copy(acc_v, out_hbm.at[wid])


@functools.cache
def _sc_call():
    return functools.partial(
        pl.kernel,
        out_type=jax.ShapeDtypeStruct((_NW, _L), jnp.float32),
        mesh=plsc.VectorSubcoreMesh(core_axis_name="c", subcore_axis_name="s"),
        scratch_types=[
            pltpu.VMEM((_RPW,), jnp.int32),
            pltpu.VMEM((_L,), jnp.float32),
        ],
    )(_sc_body)


_BR = 4096
_GRID = -(-_V // _BR)


def _tc_body(t_ref, tc_ref, x_ref, out_ref):
    j = pl.program_id(0)
    xb = x_ref[...]
    w = jnp.where(tc_ref[...] != _PAD, 1.0, 0.0)

    @pl.when(j == 0)
    def _():
        row0 = lax.dot_general(xb[0:1, :], w, (((1,), (0,)), ((), ())),
                               preferred_element_type=jnp.float32)
        out_ref[...] = (_SMOOTH_VAL * row0[0, 0]).reshape(1, 1)

    vcol = j * _BR + lax.broadcasted_iota(jnp.int32, (_BR, _N), 0)
    y = xb * jnp.where(vcol == t_ref[...], -_CONF, -_SMOOTH_VAL)
    vec = lax.dot_general(y, w, (((1,), (0,)), ((), ())),
                          preferred_element_type=jnp.float32)
    vrow = j * _BR + lax.broadcasted_iota(jnp.int32, (_BR, 1), 0)
    out_ref[...] += jnp.sum(jnp.where(vrow < _V, vec, 0.0)).reshape(1, 1)


_tc_call = pl.pallas_call(
    _tc_body,
    grid=(_GRID,),
    in_specs=[
        pl.BlockSpec((1, _N), lambda j: (0, 0)),
        pl.BlockSpec((_N, 1), lambda j: (0, 0)),
        pl.BlockSpec((_BR, _N), lambda j: (j, 0)),
    ],
    out_specs=pl.BlockSpec((1, 1), lambda j: (0, 0)),
    out_shape=jax.ShapeDtypeStruct((1, 1), jnp.float32),
)


def kernel(x, target):
    assert x.shape == (_N, _V)
    t32 = target.astype(jnp.int32)
    k_partials = _sc_call()(t32)
    tc_out = _tc_call(t32.reshape(1, _N), t32.reshape(_N, 1),
                      jnp.swapaxes(x, 0, 1))
    return tc_out[0, 0] + jnp.sum(k_partials)
